# baseline (speedup 1.0000x reference)
import functools

import jax
import jax.numpy as jnp
from jax import lax
from jax.experimental import pallas as pl
from jax.experimental.pallas import tpu as pltpu
from jax.experimental.pallas import tpu_sc as plsc

N = 10000
K = 32
D = 128

NW = 32
NPAD = 10240
NPW = NPAD // NW
C = 2
G = C * K
NSTEPS = NPW // C
W2D_ROWS = NPAD * K // 128


def _sc_aggregate(features, idx_packed, w_flat):
    mesh = plsc.VectorSubcoreMesh(core_axis_name="c", subcore_axis_name="s")

    @functools.partial(
        pl.kernel,
        out_type=jax.ShapeDtypeStruct((NPAD, D), jnp.float32),
        mesh=mesh,
        scratch_types=[
            pltpu.VMEM((2, G // 2), jnp.int32),
            pltpu.VMEM((2, G), jnp.int32),
            pltpu.VMEM((2, G), jnp.float32),
            pltpu.VMEM((2, G, D), jnp.float32),
            pltpu.VMEM((2, C, D), jnp.float32),
            pltpu.VMEM_SHARED((N, D), jnp.float32),
            pltpu.SemaphoreType.DMA,
            pltpu.SemaphoreType.DMA,
            pltpu.SemaphoreType.DMA,
            pltpu.SemaphoreType.DMA,
            pltpu.SemaphoreType.DMA,
            pltpu.SemaphoreType.DMA,
            pltpu.SemaphoreType.DMA,
            pltpu.SemaphoreType.DMA,
        ],
    )
    def agg_kernel(feat_hbm, idx_hbm, w_hbm, out_hbm,
                   pk_v, ich_v, wch_v, rows_v, ob_v, feat_sp,
                   gs0, gs1, os0, os1, is0, is1, ws0, ws1):
        gsems = [gs0, gs1]
        osems = [os0, os1]
        isems = [is0, is1]
        wsems = [ws0, ws1]
        sid = lax.axis_index("s")
        wid = sid * 2 + lax.axis_index("c")
        obase = wid * NPW
        fbase = wid * NPW * K
        pbase = wid * (NPW * K // 2)

        def ich_dma(step, j):
            return pltpu.make_async_copy(
                idx_hbm.at[pl.ds(
                    pl.multiple_of(pbase + step * (G // 2), 8), G // 2)],
                pk_v.at[j], isems[j])

        def wch_dma(step, j):
            return pltpu.make_async_copy(
                w_hbm.at[pl.ds(
                    pl.multiple_of(fbase + step * G, 8), G)],
                wch_v.at[j], wsems[j])

        def gather(j):
            return pltpu.make_async_copy(
                feat_sp.at[ich_v.at[j]], rows_v.at[j], gsems[j])

        def out_dma(step, j):
            return pltpu.make_async_copy(
                ob_v.at[j], out_hbm.at[pl.ds(obase + step * C, C)], osems[j])

        def unpack_idx(j):
            for nl in range(C):
                pk = pk_v[j, pl.ds(nl * 16, 16)]
                ich_v[j, pl.ds(nl * K, 16)] = pk & jnp.int32(0xFFFF)
                ich_v[j, pl.ds(nl * K + 16, 16)] = lax.shift_right_logical(
                    pk, jnp.int32(16))

        srows = 632
        soff = pl.multiple_of(jnp.minimum(sid * srows, N - srows), 8)
        pltpu.sync_copy(feat_hbm.at[pl.ds(soff, srows)],
                        feat_sp.at[pl.ds(soff, srows)])
        for j in range(2):
            pltpu.sync_copy(
                idx_hbm.at[pl.ds(pbase + j * (G // 2), G // 2)], pk_v.at[j])
            pltpu.sync_copy(
                w_hbm.at[pl.ds(fbase + j * G, G)], wch_v.at[j])
            unpack_idx(j)
        plsc.subcore_barrier()
        for j in range(2):
            gather(j).start()

        def compute(j):
            for nl in range(C):
                wv0 = wch_v[j, pl.ds(nl * K, 16)]
                wv1 = wch_v[j, pl.ds(nl * K + 16, 16)]
                acc = [jnp.zeros((16,), jnp.float32) for _ in range(D // 16)]
                for k in range(K):
                    ws = wv0[k] if k < 16 else wv1[k - 16]
                    r = nl * K + k
                    for d in range(D // 16):
                        acc[d] = acc[d] + ws * rows_v[j, r, pl.ds(d * 16, 16)]
                for d in range(D // 16):
                    ob_v[j, nl, pl.ds(d * 16, 16)] = acc[d]

        def step_work(step, j):
            gather(j).wait()

            @pl.when(step + 2 < NSTEPS)
            def _():
                ich_dma(step + 2, j).start()

            @pl.when(step >= 2)
            def _():
                wch_dma(step, j).wait()
                out_dma(step - 2, j).wait()

            compute(j)
            out_dma(step, j).start()

            @pl.when(step + 2 < NSTEPS)
            def _():
                wch_dma(step + 2, j).start()
                ich_dma(step + 2, j).wait()
                unpack_idx(j)
                gather(j).start()

        def main_body(g, carry):
            for j in range(2):
                step_work(g * 2 + j, j)
            return carry
        lax.fori_loop(0, NSTEPS // 2, main_body, 0)

        out_dma(NSTEPS - 2, 0).wait()
        out_dma(NSTEPS - 1, 1).wait()

    return agg_kernel(features, idx_packed, w_flat)


def _tc_normalize_weights(w2d, Q):
    BR = 512

    def body(w_ref, q_ref, o_ref):
        w = w_ref[...]
        s = lax.dot_general(w, q_ref[...], (((1,), (0,)), ((), ())),
                            preferred_element_type=jnp.float32)
        z = s == 0.0
        safe = jnp.where(z, 1.0, s)
        o_ref[...] = jnp.where(z, jnp.float32(1.0 / K), w / safe)

    return pl.pallas_call(
        body,
        grid=(W2D_ROWS // BR,),
        in_specs=[
            pl.BlockSpec((BR, 128), lambda i: (i, 0)),
            pl.BlockSpec((128, 128), lambda i: (0, 0)),
        ],
        out_specs=pl.BlockSpec((BR, 128), lambda i: (i, 0)),
        out_shape=jax.ShapeDtypeStruct((W2D_ROWS, 128), jnp.float32),
    )(w2d, Q)


def _tc_linear_layernorm(agg, W, b, gamma, beta):
    BR = 1024

    def body(x_ref, w_ref, b_ref, g_ref, be_ref, o_ref):
        x = x_ref[...]
        y = lax.dot_general(x, w_ref[...], (((1,), (1,)), ((), ())),
                            preferred_element_type=jnp.float32)
        y = y + b_ref[...]
        m = jnp.mean(y, axis=-1, keepdims=True)
        dlt = y - m
        var = jnp.mean(dlt * dlt, axis=-1, keepdims=True)
        o_ref[...] = (dlt * lax.rsqrt(var + 1e-5)) * g_ref[...] + be_ref[...]

    return pl.pallas_call(
        body,
        grid=(NPAD // BR,),
        in_specs=[
            pl.BlockSpec((BR, D), lambda i: (i, 0)),
            pl.BlockSpec((D, D), lambda i: (0, 0)),
            pl.BlockSpec((1, D), lambda i: (0, 0)),
            pl.BlockSpec((1, D), lambda i: (0, 0)),
            pl.BlockSpec((1, D), lambda i: (0, 0)),
        ],
        out_specs=pl.BlockSpec((BR, D), lambda i: (i, 0)),
        out_shape=jax.ShapeDtypeStruct((NPAD, D), jnp.float32),
    )(agg, W, b.reshape(1, D), gamma.reshape(1, D), beta.reshape(1, D))


def kernel(features, neighbors, importance_weights, W, b, gamma, beta):
    idx = neighbors.astype(jnp.int32)
    pad = NPAD - N
    idx_p = jnp.pad(idx, ((0, pad), (0, 0)))
    w_p = jnp.pad(importance_weights, ((0, pad), (0, 0)), constant_values=1.0)
    blk = jnp.arange(128, dtype=jnp.int32) // K
    Q = (blk[:, None] == blk[None, :]).astype(jnp.float32)
    w_norm = _tc_normalize_weights(w_p.reshape(W2D_ROWS, 128), Q)
    idx_flat = idx_p.reshape(-1, 2)
    idx_packed = idx_flat[:, 0] | (idx_flat[:, 1] << 16)
    evens = jnp.arange(16, dtype=jnp.int32) * 2
    perm32 = jnp.concatenate([evens, evens + 1])
    w_perm = w_norm.reshape(NPAD, K)[:, perm32]
    agg = _sc_aggregate(features, idx_packed, w_perm.reshape(-1))
    out = _tc_linear_layernorm(agg, W, b, gamma, beta)
    return out[:N]


# speedup vs baseline: 1.7481x; 1.7481x over previous
import functools

import jax
import jax.numpy as jnp
from jax import lax
from jax.experimental import pallas as pl
from jax.experimental.pallas import tpu as pltpu
from jax.experimental.pallas import tpu_sc as plsc

N = 10000
K = 32
D = 128

NW = 32
NPAD = 10240
NPW = NPAD // NW
G = K
NSTEPS = NPW
W2D_ROWS = NPAD * K // 128


def _sc_aggregate(features, idx_flat, w_flat):
    mesh = plsc.VectorSubcoreMesh(core_axis_name="c", subcore_axis_name="s")

    @functools.partial(
        pl.kernel,
        out_type=jax.ShapeDtypeStruct((NPAD, D), jnp.float32),
        mesh=mesh,
        scratch_types=[
            pltpu.VMEM((2, K), jnp.int32),
            pltpu.VMEM((2, K), jnp.float32),
            pltpu.VMEM((2, G, D), jnp.float32),
            pltpu.VMEM((2, 1, D), jnp.float32),
            pltpu.VMEM_SHARED((N, D), jnp.float32),
            pltpu.SemaphoreType.DMA,
            pltpu.SemaphoreType.DMA,
            pltpu.SemaphoreType.DMA,
            pltpu.SemaphoreType.DMA,
            pltpu.SemaphoreType.DMA,
            pltpu.SemaphoreType.DMA,
            pltpu.SemaphoreType.DMA,
            pltpu.SemaphoreType.DMA,
        ],
    )
    def agg_kernel(feat_hbm, idx_hbm, w_hbm, out_hbm,
                   ich_v, wch_v, rows_v, ob_v, feat_sp,
                   gs0, gs1, os0, os1, is0, is1, ws0, ws1):
        gsems = [gs0, gs1]
        osems = [os0, os1]
        isems = [is0, is1]
        wsems = [ws0, ws1]
        sid = lax.axis_index("s")
        wid = sid * 2 + lax.axis_index("c")
        obase = wid * NPW
        fbase = wid * NPW * K

        def ich_dma(step, j):
            return pltpu.make_async_copy(
                idx_hbm.at[pl.ds(fbase + step * K, K)], ich_v.at[j],
                isems[j])

        def wch_dma(step, j):
            return pltpu.make_async_copy(
                w_hbm.at[pl.ds(fbase + step * K, K)], wch_v.at[j], wsems[j])

        def gather(j):
            return pltpu.make_async_copy(
                feat_sp.at[ich_v.at[j]], rows_v.at[j], gsems[j])

        def out_dma(step, j):
            return pltpu.make_async_copy(
                ob_v.at[j], out_hbm.at[pl.ds(obase + step, 1)], osems[j])

        srows = 632
        soff = pl.multiple_of(jnp.minimum(sid * srows, N - srows), 8)
        pltpu.sync_copy(feat_hbm.at[pl.ds(soff, srows)],
                        feat_sp.at[pl.ds(soff, srows)])
        for j in range(2):
            pltpu.sync_copy(idx_hbm.at[pl.ds(fbase + j * K, K)], ich_v.at[j])
            pltpu.sync_copy(w_hbm.at[pl.ds(fbase + j * K, K)], wch_v.at[j])
        plsc.subcore_barrier()
        for j in range(2):
            gather(j).start()

        def compute(j):
            wv0 = wch_v[j, pl.ds(0, 16)]
            wv1 = wch_v[j, pl.ds(16, 16)]
            acc = [jnp.zeros((16,), jnp.float32) for _ in range(D // 16)]
            for k in range(K):
                ws = wv0[k] if k < 16 else wv1[k - 16]
                for d in range(D // 16):
                    acc[d] = acc[d] + ws * rows_v[j, k, pl.ds(d * 16, 16)]
            for d in range(D // 16):
                ob_v[j, 0, pl.ds(d * 16, 16)] = acc[d]

        def step_work(step, j):
            gather(j).wait()

            @pl.when(step + 2 < NSTEPS)
            def _():
                ich_dma(step + 2, j).start()

            @pl.when(step >= 2)
            def _():
                wch_dma(step, j).wait()
                out_dma(step - 2, j).wait()

            compute(j)
            out_dma(step, j).start()

            @pl.when(step + 2 < NSTEPS)
            def _():
                wch_dma(step + 2, j).start()
                ich_dma(step + 2, j).wait()
                gather(j).start()

        def main_body(g, carry):
            for j in range(2):
                step_work(g * 2 + j, j)
            return carry
        lax.fori_loop(0, NSTEPS // 2, main_body, 0)

        out_dma(NSTEPS - 2, 0).wait()
        out_dma(NSTEPS - 1, 1).wait()

    return agg_kernel(features, idx_flat, w_flat)


def _tc_normalize_weights(w2d, Q):
    BR = 512

    def body(w_ref, q_ref, o_ref):
        w = w_ref[...]
        s = lax.dot_general(w, q_ref[...], (((1,), (0,)), ((), ())),
                            preferred_element_type=jnp.float32)
        z = s == 0.0
        safe = jnp.where(z, 1.0, s)
        o_ref[...] = jnp.where(z, jnp.float32(1.0 / K), w / safe)

    return pl.pallas_call(
        body,
        grid=(W2D_ROWS // BR,),
        in_specs=[
            pl.BlockSpec((BR, 128), lambda i: (i, 0)),
            pl.BlockSpec((128, 128), lambda i: (0, 0)),
        ],
        out_specs=pl.BlockSpec((BR, 128), lambda i: (i, 0)),
        out_shape=jax.ShapeDtypeStruct((W2D_ROWS, 128), jnp.float32),
    )(w2d, Q)


def _tc_linear_layernorm(agg, W, b, gamma, beta):
    BR = 1024

    def body(x_ref, w_ref, b_ref, g_ref, be_ref, o_ref):
        x = x_ref[...]
        y = lax.dot_general(x, w_ref[...], (((1,), (1,)), ((), ())),
                            preferred_element_type=jnp.float32)
        y = y + b_ref[...]
        m = jnp.mean(y, axis=-1, keepdims=True)
        dlt = y - m
        var = jnp.mean(dlt * dlt, axis=-1, keepdims=True)
        o_ref[...] = (dlt * lax.rsqrt(var + 1e-5)) * g_ref[...] + be_ref[...]

    return pl.pallas_call(
        body,
        grid=(NPAD // BR,),
        in_specs=[
            pl.BlockSpec((BR, D), lambda i: (i, 0)),
            pl.BlockSpec((D, D), lambda i: (0, 0)),
            pl.BlockSpec((1, D), lambda i: (0, 0)),
            pl.BlockSpec((1, D), lambda i: (0, 0)),
            pl.BlockSpec((1, D), lambda i: (0, 0)),
        ],
        out_specs=pl.BlockSpec((BR, D), lambda i: (i, 0)),
        out_shape=jax.ShapeDtypeStruct((NPAD, D), jnp.float32),
    )(agg, W, b.reshape(1, D), gamma.reshape(1, D), beta.reshape(1, D))


def kernel(features, neighbors, importance_weights, W, b, gamma, beta):
    idx = neighbors.astype(jnp.int32)
    pad = NPAD - N
    idx_p = jnp.pad(idx, ((0, pad), (0, 0)))
    w_p = jnp.pad(importance_weights, ((0, pad), (0, 0)), constant_values=1.0)
    blk = jnp.arange(128, dtype=jnp.int32) // K
    Q = (blk[:, None] == blk[None, :]).astype(jnp.float32)
    w_norm = _tc_normalize_weights(w_p.reshape(W2D_ROWS, 128), Q)
    agg = _sc_aggregate(features, idx_p.reshape(-1),
                        w_norm.reshape(-1))
    out = _tc_linear_layernorm(agg, W, b, gamma, beta)
    return out[:N]


# speedup vs baseline: 1.7674x; 1.0110x over previous
import functools

import jax
import jax.numpy as jnp
from jax import lax
from jax.experimental import pallas as pl
from jax.experimental.pallas import tpu as pltpu
from jax.experimental.pallas import tpu_sc as plsc

N = 10000
K = 32
D = 128

NW = 32
NPAD = 10240
NPW = NPAD // NW
G = K
NSTEPS = NPW
W2D_ROWS = NPAD * K // 128


def _sc_aggregate(features, idx_flat, w_flat):
    mesh = plsc.VectorSubcoreMesh(core_axis_name="c", subcore_axis_name="s")

    @functools.partial(
        pl.kernel,
        out_type=jax.ShapeDtypeStruct((2 * NPAD, D), jnp.float32),
        mesh=mesh,
        scratch_types=[
            pltpu.VMEM((2, K), jnp.int32),
            pltpu.VMEM((2, K), jnp.float32),
            pltpu.VMEM((2, G, D), jnp.float32),
            pltpu.VMEM((2, 1, D), jnp.float32),
            pltpu.VMEM((2, 1, D), jnp.float32),
            pltpu.VMEM_SHARED((N, D), jnp.float32),
            pltpu.SemaphoreType.DMA,
            pltpu.SemaphoreType.DMA,
            pltpu.SemaphoreType.DMA,
            pltpu.SemaphoreType.DMA,
            pltpu.SemaphoreType.DMA,
            pltpu.SemaphoreType.DMA,
            pltpu.SemaphoreType.DMA,
            pltpu.SemaphoreType.DMA,
            pltpu.SemaphoreType.DMA,
            pltpu.SemaphoreType.DMA,
        ],
    )
    def agg_kernel(feat_hbm, idx_hbm, w_hbm, out_hbm,
                   ich_v, wch_v, rows_v, ob_v, ub_v, feat_sp,
                   gs0, gs1, os0, os1, us0, us1, is0, is1, ws0, ws1):
        gsems = [gs0, gs1]
        osems = [os0, os1]
        usems = [us0, us1]
        isems = [is0, is1]
        wsems = [ws0, ws1]
        sid = lax.axis_index("s")
        wid = sid * 2 + lax.axis_index("c")
        obase = wid * NPW
        fbase = wid * NPW * K

        def ich_dma(step, j):
            return pltpu.make_async_copy(
                idx_hbm.at[pl.ds(fbase + step * K, K)], ich_v.at[j],
                isems[j])

        def wch_dma(step, j):
            return pltpu.make_async_copy(
                w_hbm.at[pl.ds(fbase + step * K, K)], wch_v.at[j], wsems[j])

        def gather(j):
            return pltpu.make_async_copy(
                feat_sp.at[ich_v.at[j]], rows_v.at[j], gsems[j])

        def out_dma(step, j):
            return pltpu.make_async_copy(
                ob_v.at[j], out_hbm.at[pl.ds(obase + step, 1)], osems[j])

        def uout_dma(step, j):
            return pltpu.make_async_copy(
                ub_v.at[j], out_hbm.at[pl.ds(NPAD + obase + step, 1)],
                usems[j])

        srows = 632
        soff = pl.multiple_of(jnp.minimum(sid * srows, N - srows), 8)
        pltpu.sync_copy(feat_hbm.at[pl.ds(soff, srows)],
                        feat_sp.at[pl.ds(soff, srows)])
        for j in range(2):
            pltpu.sync_copy(idx_hbm.at[pl.ds(fbase + j * K, K)], ich_v.at[j])
            pltpu.sync_copy(w_hbm.at[pl.ds(fbase + j * K, K)], wch_v.at[j])
        plsc.subcore_barrier()
        for j in range(2):
            gather(j).start()

        def compute(j):
            wv0 = wch_v[j, pl.ds(0, 16)]
            wv1 = wch_v[j, pl.ds(16, 16)]
            acc = [jnp.zeros((16,), jnp.float32) for _ in range(D // 16)]
            uacc = [jnp.zeros((16,), jnp.float32) for _ in range(D // 16)]
            for k in range(K):
                ws = wv0[k] if k < 16 else wv1[k - 16]
                for d in range(D // 16):
                    row = rows_v[j, k, pl.ds(d * 16, 16)]
                    acc[d] = acc[d] + ws * row
                    uacc[d] = uacc[d] + row
            for d in range(D // 16):
                ob_v[j, 0, pl.ds(d * 16, 16)] = acc[d]
                ub_v[j, 0, pl.ds(d * 16, 16)] = uacc[d]

        def step_work(step, j):
            gather(j).wait()

            @pl.when(step + 2 < NSTEPS)
            def _():
                ich_dma(step + 2, j).start()

            @pl.when(step >= 2)
            def _():
                wch_dma(step, j).wait()
                out_dma(step - 2, j).wait()
                uout_dma(step - 2, j).wait()

            compute(j)
            out_dma(step, j).start()
            uout_dma(step, j).start()

            @pl.when(step + 2 < NSTEPS)
            def _():
                wch_dma(step + 2, j).start()
                ich_dma(step + 2, j).wait()
                gather(j).start()

        def main_body(g, carry):
            for j in range(2):
                step_work(g * 2 + j, j)
            return carry
        lax.fori_loop(0, NSTEPS // 2, main_body, 0)

        out_dma(NSTEPS - 2, 0).wait()
        out_dma(NSTEPS - 1, 1).wait()
        uout_dma(NSTEPS - 2, 0).wait()
        uout_dma(NSTEPS - 1, 1).wait()

    return agg_kernel(features, idx_flat, w_flat)


def _tc_epilogue(agg2, w_p, W, b, gamma, beta):
    BR = 1024
    NB = NPAD // BR

    def body(a_ref, u_ref, w_ref, wm_ref, b_ref, g_ref, be_ref, o_ref):
        s = jnp.sum(w_ref[...], axis=-1, keepdims=True)
        z = s == 0.0
        agg = jnp.where(z, u_ref[...] * jnp.float32(1.0 / K),
                        a_ref[...] / jnp.where(z, 1.0, s))
        y = lax.dot_general(agg, wm_ref[...], (((1,), (1,)), ((), ())),
                            preferred_element_type=jnp.float32)
        y = y + b_ref[...]
        m = jnp.mean(y, axis=-1, keepdims=True)
        dlt = y - m
        var = jnp.mean(dlt * dlt, axis=-1, keepdims=True)
        o_ref[...] = (dlt * lax.rsqrt(var + 1e-5)) * g_ref[...] + be_ref[...]

    return pl.pallas_call(
        body,
        grid=(NB,),
        in_specs=[
            pl.BlockSpec((BR, D), lambda i: (i, 0)),
            pl.BlockSpec((BR, D), lambda i: (i + NB, 0)),
            pl.BlockSpec((BR, K), lambda i: (i, 0)),
            pl.BlockSpec((D, D), lambda i: (0, 0)),
            pl.BlockSpec((1, D), lambda i: (0, 0)),
            pl.BlockSpec((1, D), lambda i: (0, 0)),
            pl.BlockSpec((1, D), lambda i: (0, 0)),
        ],
        out_specs=pl.BlockSpec((BR, D), lambda i: (i, 0)),
        out_shape=jax.ShapeDtypeStruct((NPAD, D), jnp.float32),
    )(agg2, agg2, w_p, W, b.reshape(1, D), gamma.reshape(1, D),
      beta.reshape(1, D))


def kernel(features, neighbors, importance_weights, W, b, gamma, beta):
    idx = neighbors.astype(jnp.int32)
    pad = NPAD - N
    idx_p = jnp.pad(idx, ((0, pad), (0, 0)))
    w_p = jnp.pad(importance_weights, ((0, pad), (0, 0)), constant_values=1.0)
    agg2 = _sc_aggregate(features, idx_p.reshape(-1), w_p.reshape(-1))
    out = _tc_epilogue(agg2, w_p, W, b, gamma, beta)
    return out[:N]
